# baseline (speedup 1.0000x reference)
import functools

import jax
import jax.numpy as jnp
from jax.experimental import pallas as pl
from jax.experimental.pallas import tpu as pltpu

_HIDDEN = 4096
_EXPERTS = 128
_TOPK = 8
_TOKENS = 8192
_CH = 512
_NCH = _TOKENS // _CH
_BUFS = 4


def _topk_vals(scores, brow):
    vals = scores + brow
    lanef = jax.lax.broadcasted_iota(jnp.int32, (_CH, _EXPERTS), 1).astype(
        jnp.float32
    )
    crow = lanef + brow
    m_cols = []
    c_cols = []
    for _ in range(_TOPK):
        m = jnp.max(vals, axis=1, keepdims=True)
        eq = vals == m
        c = jnp.min(jnp.where(eq, crow, jnp.inf), axis=1, keepdims=True)
        vals = jnp.where(crow == c, -jnp.inf, vals)
        m_cols.append(m)
        c_cols.append(c)
    mcat = jnp.concatenate(m_cols, axis=1)
    ccat = jnp.concatenate(c_cols, axis=1)
    idxf = jnp.floor(ccat + 0.5)
    ws = mcat - (ccat - idxf)
    ws = ws / (jnp.sum(ws, axis=1, keepdims=True) + 1e-20)
    return idxf.astype(jnp.int32), ws


def _router(hs_ref, w_ref, b_ref, idx_ref, wgt_ref, bufs, sems):
    brow = b_ref[...]
    w = w_ref[...]

    def _copy(c):
        return pltpu.make_async_copy(
            hs_ref.at[pl.ds(c * _CH, _CH), :],
            bufs.at[c % _BUFS],
            sems.at[c % _BUFS],
        )

    for c in range(_BUFS):
        _copy(c).start()
    for c in range(_NCH):
        _copy(c).wait()
        logits = jnp.dot(
            bufs[c % _BUFS], w, preferred_element_type=jnp.float32
        )
        if c + _BUFS < _NCH:
            _copy(c + _BUFS).start()
        idxs, ws = _topk_vals(jax.nn.sigmoid(logits), brow)
        idx_ref[pl.ds(c * _CH, _CH), :] = idxs
        wgt_ref[pl.ds(c * _CH, _CH), :] = ws


@functools.partial(jax.jit)
def kernel(hidden_states, weight, e_score_correction_bias):
    hs = hidden_states.reshape(-1, _HIDDEN)
    wt = weight.astype(jnp.float32).T
    bias = e_score_correction_bias.reshape(1, _EXPERTS)
    idxs, ws = pl.pallas_call(
        _router,
        in_specs=[
            pl.BlockSpec(memory_space=pltpu.MemorySpace.HBM),
            pl.BlockSpec((_HIDDEN, _EXPERTS), lambda: (0, 0)),
            pl.BlockSpec((1, _EXPERTS), lambda: (0, 0)),
        ],
        out_specs=[
            pl.BlockSpec((_TOKENS, _TOPK), lambda: (0, 0)),
            pl.BlockSpec((_TOKENS, _TOPK), lambda: (0, 0)),
        ],
        out_shape=[
            jax.ShapeDtypeStruct((_TOKENS, _TOPK), jnp.int32),
            jax.ShapeDtypeStruct((_TOKENS, _TOPK), jnp.float32),
        ],
        scratch_shapes=[
            pltpu.VMEM((_BUFS, _CH, _HIDDEN), jnp.float32),
            pltpu.SemaphoreType.DMA((_BUFS,)),
        ],
    )(hs, wt, bias)
    return idxs, ws


# speedup vs baseline: 1.3091x; 1.3091x over previous
import functools

import jax
import jax.numpy as jnp
from jax.experimental import pallas as pl

_HIDDEN = 4096
_EXPERTS = 128
_TOPK = 8
_TOKENS = 8192
_TB = 1024


def _router_block(hs_ref, w_ref, b_ref, idx_ref, wgt_ref):
    logits = jnp.dot(hs_ref[...], w_ref[...], preferred_element_type=jnp.float32)
    scores = jax.nn.sigmoid(logits)
    vals = scores + b_ref[...]
    lanef = jax.lax.broadcasted_iota(jnp.int32, (_TB, _EXPERTS), 1).astype(
        jnp.float32
    )
    crow = lanef + b_ref[...]
    idx_cols = []
    w_cols = []
    for _ in range(_TOPK):
        m = jnp.max(vals, axis=1, keepdims=True)
        eq = vals == m
        c = jnp.min(jnp.where(eq, crow, jnp.inf), axis=1, keepdims=True)
        idxf = jnp.floor(c + 0.5)
        w = m - (c - idxf)
        vals = jnp.where(crow == c, -jnp.inf, vals)
        idx_cols.append(idxf)
        w_cols.append(w)
    idxs = jnp.concatenate(idx_cols, axis=1).astype(jnp.int32)
    ws = jnp.concatenate(w_cols, axis=1)
    ws = ws / (jnp.sum(ws, axis=1, keepdims=True) + 1e-20)
    idx_ref[...] = idxs
    wgt_ref[...] = ws


@functools.partial(jax.jit)
def kernel(hidden_states, weight, e_score_correction_bias):
    hs = hidden_states.reshape(-1, _HIDDEN)
    wt = weight.astype(jnp.float32).T
    bias = e_score_correction_bias.reshape(1, _EXPERTS)
    grid = (_TOKENS // _TB,)
    idxs, ws = pl.pallas_call(
        _router_block,
        grid=grid,
        in_specs=[
            pl.BlockSpec((_TB, _HIDDEN), lambda i: (i, 0)),
            pl.BlockSpec((_HIDDEN, _EXPERTS), lambda i: (0, 0)),
            pl.BlockSpec((1, _EXPERTS), lambda i: (0, 0)),
        ],
        out_specs=[
            pl.BlockSpec((_TB, _TOPK), lambda i: (i, 0)),
            pl.BlockSpec((_TB, _TOPK), lambda i: (i, 0)),
        ],
        out_shape=[
            jax.ShapeDtypeStruct((_TOKENS, _TOPK), jnp.int32),
            jax.ShapeDtypeStruct((_TOKENS, _TOPK), jnp.float32),
        ],
    )(hs, wt, bias)
    return idxs, ws


# speedup vs baseline: 1.3119x; 1.0021x over previous
import functools

import jax
import jax.numpy as jnp
from jax.experimental import pallas as pl

_HIDDEN = 4096
_EXPERTS = 128
_TOPK = 8
_TOKENS = 8192
_TB = 1024
_HH = _HIDDEN // 2


def _router_block(hs0_ref, hs1_ref, w0_ref, w1_ref, b_ref, idx_ref, wgt_ref):
    logits = jnp.dot(
        hs0_ref[...], w0_ref[...], preferred_element_type=jnp.float32
    ) + jnp.dot(hs1_ref[...], w1_ref[...], preferred_element_type=jnp.float32)
    scores = jax.nn.sigmoid(logits)
    vals = scores + b_ref[...]
    lanef = jax.lax.broadcasted_iota(jnp.int32, (_TB, _EXPERTS), 1).astype(
        jnp.float32
    )
    crow = lanef + b_ref[...]
    idx_cols = []
    w_cols = []
    for _ in range(_TOPK):
        m = jnp.max(vals, axis=1, keepdims=True)
        eq = vals == m
        c = jnp.min(jnp.where(eq, crow, jnp.inf), axis=1, keepdims=True)
        idxf = jnp.floor(c + 0.5)
        w = m - (c - idxf)
        vals = jnp.where(crow == c, -jnp.inf, vals)
        idx_cols.append(idxf)
        w_cols.append(w)
    idxs = jnp.concatenate(idx_cols, axis=1).astype(jnp.int32)
    ws = jnp.concatenate(w_cols, axis=1)
    ws = ws / (jnp.sum(ws, axis=1, keepdims=True) + 1e-20)
    idx_ref[...] = idxs
    wgt_ref[...] = ws


@functools.partial(jax.jit)
def kernel(hidden_states, weight, e_score_correction_bias):
    hs = hidden_states.reshape(-1, _HIDDEN)
    wt = weight.astype(jnp.float32).T
    bias = e_score_correction_bias.reshape(1, _EXPERTS)
    grid = (_TOKENS // _TB,)
    idxs, ws = pl.pallas_call(
        _router_block,
        grid=grid,
        in_specs=[
            pl.BlockSpec((_TB, _HH), lambda i: (i, 0)),
            pl.BlockSpec((_TB, _HH), lambda i: (i, 1)),
            pl.BlockSpec((_HH, _EXPERTS), lambda i: (0, 0)),
            pl.BlockSpec((_HH, _EXPERTS), lambda i: (1, 0)),
            pl.BlockSpec((1, _EXPERTS), lambda i: (0, 0)),
        ],
        out_specs=[
            pl.BlockSpec((_TB, _TOPK), lambda i: (i, 0)),
            pl.BlockSpec((_TB, _TOPK), lambda i: (i, 0)),
        ],
        out_shape=[
            jax.ShapeDtypeStruct((_TOKENS, _TOPK), jnp.int32),
            jax.ShapeDtypeStruct((_TOKENS, _TOPK), jnp.float32),
        ],
    )(hs, hs, wt, wt, bias)
    return idxs, ws
